# baseline (speedup 1.0000x reference)
import functools
import math

import jax
import jax.numpy as jnp
from jax import lax
from jax.experimental import pallas as pl
from jax.experimental.pallas import tpu as pltpu
from jax.experimental.pallas import tpu_sc as plsc

D_MODEL = 64
SCALE = math.sqrt(D_MODEL)

NC = 2
NS = 16
NW = NC * NS
LANES = 16
NBUF = 4


@functools.partial(jax.jit, static_argnames=("n_seq", "seq_len"))
def _embed(idx_flat, tab2, *, n_seq, seq_len):
    seq_per_w = n_seq // NW
    n_idx_w = seq_per_w * seq_len
    sl_pad = (seq_len + LANES - 1) // LANES * LANES
    n_full = seq_len // LANES
    n_tail = seq_len - n_full * LANES
    mesh = plsc.VectorSubcoreMesh(core_axis_name="c", subcore_axis_name="s")

    @functools.partial(
        pl.kernel,
        mesh=mesh,
        out_type=jax.ShapeDtypeStruct((n_seq, seq_len, D_MODEL), jnp.float32),
        scratch_types=[
            pltpu.VMEM((n_idx_w + sl_pad - seq_len,), jnp.int32),
            pltpu.VMEM((NBUF, sl_pad), jnp.int32),
            pltpu.VMEM((NBUF, seq_len, 2 * D_MODEL), jnp.float32),
            pltpu.SemaphoreType.DMA,
            *([pltpu.SemaphoreType.DMA] * NBUF),
            *([pltpu.SemaphoreType.DMA] * NBUF),
        ],
        compiler_params=pltpu.CompilerParams(use_tc_tiling_on_sc=False),
    )
    def k(idx_hbm, tab_hbm, out_hbm, idx_v, idxg_v, rows_v, isem, gs0, gs1,
          gs2, gs3, ss0, ss1, ss2, ss3):
        gsem = (gs0, gs1, gs2, gs3)
        ssem = (ss0, ss1, ss2, ss3)
        wid = lax.axis_index("s") * NC + lax.axis_index("c")
        seq0 = wid * seq_per_w

        pltpu.async_copy(
            idx_hbm.at[pl.ds(seq0 * seq_len, n_idx_w)],
            idx_v.at[pl.ds(0, n_idx_w)], isem).wait()

        def gather(c, b):
            return pltpu.make_async_copy(
                tab_hbm.at[idxg_v.at[b, pl.ds(0, seq_len)]], rows_v.at[b],
                gsem[b])

        def scatter(c, b):
            return pltpu.make_async_copy(
                rows_v.at[b, :, pl.ds(0, D_MODEL)], out_hbm.at[seq0 + c],
                ssem[b])

        def prep_idx(c, b):
            @pl.loop(0, sl_pad // LANES)
            def _shift(g):
                sl = pl.ds(g * LANES, LANES)
                idxg_v[b, sl] = lax.shift_right_logical(
                    idx_v[pl.ds(c * seq_len + g * LANES, LANES)], 1)

        for b in range(2):
            prep_idx(b, b)
            gather(b, b).start()

        @pl.loop(0, seq_per_w, step=NBUF)
        def _outer(t):
            for b in range(NBUF):
                c = t + b
                f = (b + 2) % NBUF
                cn = c + 2

                @pl.when(cn < seq_per_w)
                def _prefetch():
                    @pl.when(cn >= NBUF)
                    def _drain():
                        scatter(cn - NBUF, f).wait()

                    prep_idx(cn, f)
                    gather(cn, f).start()

                gather(c, b).wait()
                buf = rows_v.at[b]

                def scale_rows(r0, nrows):
                    hv = lax.bitwise_and(
                        idx_v[pl.ds(c * seq_len + r0, LANES)], 1) * D_MODEL
                    for rr in range(nrows):
                        off = hv[rr]
                        r = r0 + rr
                        for j in range(D_MODEL // LANES):
                            src = pl.ds(off + j * LANES, LANES)
                            dst = pl.ds(j * LANES, LANES)
                            buf[r, dst] = buf[r, src] * SCALE

                @pl.loop(0, n_full)
                def _scale(g):
                    scale_rows(g * LANES, LANES)

                if n_tail:
                    scale_rows(n_full * LANES, n_tail)

                scatter(c, b).start()

        for b in range(NBUF):
            scatter(seq_per_w - NBUF + b, b).wait()

    return k(idx_flat, tab2)


def kernel(input_ids, table):
    n_seq, seq_len = input_ids.shape
    idx_flat = input_ids.reshape(-1).astype(jnp.int32)
    tab2 = table.reshape(table.shape[0] // 2, 2 * D_MODEL)
    return _embed(idx_flat, tab2, n_seq=n_seq, seq_len=seq_len)


# speedup vs baseline: 1.5549x; 1.5549x over previous
import functools
import math

import jax
import jax.numpy as jnp
from jax import lax
from jax.experimental import pallas as pl
from jax.experimental.pallas import tpu as pltpu
from jax.experimental.pallas import tpu_sc as plsc

D_MODEL = 64
SCALE = math.sqrt(D_MODEL)

NC = 2
NS = 16
NW = NC * NS
LANES = 16
NBUF = 4


@functools.partial(jax.jit, static_argnames=("n_seq", "seq_len"))
def _embed(idx_flat, tab_pad, *, n_seq, seq_len):
    seq_per_w = n_seq // NW
    n_idx_w = seq_per_w * seq_len
    mesh = plsc.VectorSubcoreMesh(core_axis_name="c", subcore_axis_name="s")

    @functools.partial(
        pl.kernel,
        mesh=mesh,
        out_type=jax.ShapeDtypeStruct((n_seq, seq_len, D_MODEL), jnp.float32),
        scratch_types=[
            pltpu.VMEM((n_idx_w,), jnp.int32),
            pltpu.VMEM((2, seq_len, 2 * D_MODEL), jnp.float32),
            pltpu.VMEM((2, seq_len, D_MODEL), jnp.float32),
            pltpu.SemaphoreType.DMA,
            *([pltpu.SemaphoreType.DMA] * 2),
            *([pltpu.SemaphoreType.DMA] * 2),
        ],
    )
    def k(idx_hbm, tab_hbm, out_hbm, idx_v, in_v, out_v, isem, gs0, gs1,
          ss0, ss1):
        gsem = (gs0, gs1)
        ssem = (ss0, ss1)
        wid = lax.axis_index("s") * NC + lax.axis_index("c")
        seq0 = wid * seq_per_w

        pltpu.async_copy(
            idx_hbm.at[pl.ds(seq0 * seq_len, n_idx_w)], idx_v, isem).wait()

        def gather(c, b):
            return pltpu.make_async_copy(
                tab_hbm.at[idx_v.at[pl.ds(c * seq_len, seq_len)]],
                in_v.at[b], gsem[b])

        def scatter(c, b):
            return pltpu.make_async_copy(
                out_v.at[b], out_hbm.at[seq0 + c], ssem[b])

        gather(0, 0).start()
        gather(1, 1).start()

        @pl.loop(0, seq_per_w, step=2)
        def _outer(t):
            for b in range(2):
                c = t + b
                gather(c, b).wait()

                @pl.when(c >= 2)
                def _drain():
                    scatter(c - 2, b).wait()

                src = in_v.at[b]
                dst = out_v.at[b]

                @pl.loop(0, seq_len)
                def _scale(r):
                    for j in range(D_MODEL // LANES):
                        sl = pl.ds(j * LANES, LANES)
                        dst[r, sl] = src[r, sl] * SCALE

                scatter(c, b).start()

                @pl.when(c + 2 < seq_per_w)
                def _prefetch():
                    gather(c + 2, b).start()

        for b in range(2):
            scatter(seq_per_w - 2 + b, b).wait()

    return k(idx_flat, tab_pad)


def kernel(input_ids, table):
    n_seq, seq_len = input_ids.shape
    idx_flat = input_ids.reshape(-1).astype(jnp.int32)
    tab_pad = jnp.pad(table, ((0, 0), (0, D_MODEL)))
    return _embed(idx_flat, tab_pad, n_seq=n_seq, seq_len=seq_len)
